# Initial kernel scaffold; baseline (speedup 1.0000x reference)
import jax
import jax.numpy as jnp
from jax.experimental import pallas as pl


def kernel(x, node_embedding, conv1_w, conv1_b, conv2_w, conv2_b, skip_w, skip_b, res_w, res_b, re1_w, re1_b, re2_w, re2_b, re3_w, re3_b, re4_w, re4_b):
    raise NotImplementedError("write your pallas kernel here")


# speedup vs baseline: 3.2617x; 3.2617x over previous
import jax
import jax.numpy as jnp
import numpy as np
from jax.experimental import pallas as pl
from jax.experimental.pallas import tpu as pltpu

F32 = jnp.float32
_B, _C, _N, _T = 4, 32, 256, 288
_P = 12
_KW = 12
_PERIOD, _SHORT, _LONG = 24, 12, 48
_RK = 5
_TE = _T + _P
_TO = _TE + 1 - _KW + 1
_NB = 8
_G = 13


def _win_matrix(L):
    s = np.arange(_T)[:, None]
    t = np.maximum(np.arange(_T)[None, :], L - 1)
    return (((s <= t) & (s >= t - L + 1)).astype(np.float32) / L)


def _season_matrix(P):
    s = np.arange(_T)[:, None]
    t = np.arange(_T)[None, :]
    return ((s % P) == (t % P)).astype(np.float32) * (P / _T)


_M48 = _win_matrix(_LONG)
_S24 = _season_matrix(_PERIOD)
_M12 = _win_matrix(_SHORT)


def _adj_kernel(emb_ref, adj_ref):
    e = emb_ref[...]
    logits = jax.lax.dot_general(e, e, (((1,), (1,)), ((), ())),
                                 preferred_element_type=F32)
    r = jax.lax.broadcasted_iota(jnp.int32, (_N, _N), 0)
    c = jax.lax.broadcasted_iota(jnp.int32, (_N, _N), 1)
    logits = logits - jnp.where(r == c, 10.0, 0.0).astype(F32)
    logits = logits - jnp.max(logits, axis=-1, keepdims=True)
    p = jnp.exp(logits)
    adj_ref[...] = p / jnp.sum(p, axis=-1, keepdims=True)


def _chain_kernel(x_ref, m48_ref, s24_ref, m12_ref, adj_ref,
                  x1_ref, ltm_ref, lts_ref,
                  x2_ref, sem_ref, ses_ref,
                  x3_ref, stm_ref, sts_ref,
                  x4_ref, snm_ref, sns_ref):
    xb = x_ref[0]

    def tnorm(v, M):
        m = jnp.dot(v, M, preferred_element_type=F32)
        q = jnp.dot(v * v, M, preferred_element_type=F32)
        var = q - m * m + 1e-5
        out = (v - m) * jax.lax.rsqrt(var + 1.0)
        return out, m, jnp.sqrt(var)

    x1, m1, s1 = tnorm(xb, m48_ref[...])
    x2, m2, s2 = tnorm(x1, s24_ref[...])
    x3, m3, s3 = tnorm(x2, m12_ref[...])

    a = adj_ref[...]
    ms = jnp.dot(a, x3, preferred_element_type=F32)
    qs = jnp.dot(a, x3 * x3, preferred_element_type=F32)
    vs = qs - ms * ms + 1e-5
    x4 = (x3 - ms) * jax.lax.rsqrt(vs + 1.0)
    ss = jnp.sqrt(vs)

    def extc(m):
        return jnp.concatenate(
            [m, jnp.broadcast_to(m[:, _T - 1:_T], (_N, _P))], axis=-1)

    def exts(m):
        return jnp.concatenate([m, m[:, :_P]], axis=-1)

    x1_ref[0] = x1
    ltm_ref[0] = extc(m1)
    lts_ref[0] = extc(s1)
    x2_ref[0] = x2
    sem_ref[0] = exts(m2)
    ses_ref[0] = exts(s2)
    x3_ref[0] = x3
    stm_ref[0] = extc(m3)
    sts_ref[0] = extc(s3)
    x4_ref[0] = x4
    snm_ref[0] = extc(ms)
    sns_ref[0] = extc(ss)


def _rex_kernel(x_ref, w_ref, b_ref, o_ref):
    o_ref[0, 0] = (jnp.dot(x_ref[0, 0], w_ref[0],
                           preferred_element_type=F32) + b_ref[0, 0])


def _conv_kernel(x_ref, x1_ref, ltm_ref, lts_ref, x2_ref, sem_ref, ses_ref,
                 x3_ref, stm_ref, sts_ref, x4_ref, snm_ref, sns_ref,
                 p1_ref, p2_ref, p3_ref, p4_ref,
                 w_ref, bc_ref, rw_ref, rb_ref, sw_ref, sb_ref,
                 xz_ref, s_ref):
    def body(ref):
        b = ref[0]
        return jnp.concatenate(
            [b, jnp.broadcast_to(b[:, :, _T - 1:_T], (_C, _NB, _P))], axis=-1)

    def withp(ref, pref):
        return jnp.concatenate([ref[0], pref[0]], axis=-1)

    groups = [body(x_ref), withp(x1_ref, p1_ref), ltm_ref[0], lts_ref[0],
              withp(x2_ref, p2_ref), sem_ref[0], ses_ref[0],
              withp(x3_ref, p3_ref), stm_ref[0], sts_ref[0],
              withp(x4_ref, p4_ref), snm_ref[0], sns_ref[0]]
    slab = jnp.concatenate(groups, axis=0)
    slab = jnp.concatenate(
        [jnp.zeros((_G * _C, _NB, 1), F32), slab], axis=-1)

    y = jnp.einsum('oc,cnt->ont', w_ref[...], slab,
                   preferred_element_type=F32)
    acc = jnp.zeros((64, _NB, _TO), F32)
    for k in range(_KW):
        acc = acc + y[64 * k:64 * (k + 1), :, k:k + _TO]
    acc = acc + bc_ref[0, 0][:, None, None]

    prod = acc[:_C] * acc[_C:]
    xz = jnp.einsum('oc,cnt->ont', rw_ref[...], prod[:, :, :_TO - _P],
                    preferred_element_type=F32) + rb_ref[0, 0][:, None, None]
    sk = jnp.einsum('oc,cnt->ont', sw_ref[...], prod[:, :, _TO - _P:],
                    preferred_element_type=F32) + sb_ref[0, 0][:, None, None]
    xz_ref[0] = xz
    s_ref[0] = sk


def _aux_kernel(slab_ref, w_ref, bc_ref, sw_ref, sb_ref, o_ref):
    slab = slab_ref[0]
    y = jnp.einsum('oc,ctn->otn', w_ref[...], slab,
                   preferred_element_type=F32)
    acc = jnp.zeros((64, _P, _N), F32)
    for k in range(_KW):
        acc = acc + y[64 * k:64 * (k + 1), k:k + _P, :]
    acc = acc + bc_ref[0, 0][:, None, None]
    prod = acc[:_C] * acc[_C:]
    o_ref[0] = jnp.einsum('oc,ctn->otn', sw_ref[...], prod,
                          preferred_element_type=F32) + sb_ref[0, 0][:, None, None]


def kernel(x, node_embedding, conv1_w, conv1_b, conv2_w, conv2_b,
           skip_w, skip_b, res_w, res_b,
           re1_w, re1_b, re2_w, re2_b, re3_w, re3_b, re4_w, re4_b):
    adj = pl.pallas_call(
        _adj_kernel,
        out_shape=jax.ShapeDtypeStruct((_N, _N), F32),
    )(node_embedding)

    xf = x.reshape(_B * _C, _N, _T)
    big = jax.ShapeDtypeStruct((_B * _C, _N, _T), F32)
    ext = jax.ShapeDtypeStruct((_B * _C, _N, _TE), F32)
    row = lambda t: pl.BlockSpec((1, _N, t), lambda i: (i, 0, 0))
    full2 = lambda a, b: pl.BlockSpec((a, b), lambda i: (0, 0))
    outs = pl.pallas_call(
        _chain_kernel,
        grid=(_B * _C,),
        in_specs=[row(_T), full2(_T, _T), full2(_T, _T), full2(_T, _T),
                  full2(_N, _N)],
        out_specs=[row(_T), row(_TE), row(_TE),
                   row(_T), row(_TE), row(_TE),
                   row(_T), row(_TE), row(_TE),
                   row(_T), row(_TE), row(_TE)],
        out_shape=[big, ext, ext, big, ext, ext, big, ext, ext,
                   big, ext, ext],
        compiler_params=pltpu.CompilerParams(
            dimension_semantics=("parallel",)),
    )(xf, jnp.asarray(_M48), jnp.asarray(_S24), jnp.asarray(_M12), adj)
    x1, ltm, lts, x2, sem, ses, x3, stm, sts, x4, snm, sns = outs

    tails = jnp.stack([x1, x2, x3, x4]).reshape(
        4, _B, _C, _N, _T)[..., _T - _RK:]
    xt = tails.transpose(0, 1, 3, 2, 4).reshape(4, _B, _N, _C * _RK)
    wt = jnp.stack([re1_w, re2_w, re3_w, re4_w])[:, :, :, 0, :].transpose(
        0, 2, 3, 1).reshape(4, _C * _RK, _P * _C)
    bt = jnp.stack([re1_b, re2_b, re3_b, re4_b]).reshape(4, 1, _P * _C)
    yo = pl.pallas_call(
        _rex_kernel,
        grid=(4, _B),
        in_specs=[pl.BlockSpec((1, 1, _N, _C * _RK), lambda l, b: (l, b, 0, 0)),
                  pl.BlockSpec((1, _C * _RK, _P * _C), lambda l, b: (l, 0, 0)),
                  pl.BlockSpec((1, 1, _P * _C), lambda l, b: (l, 0, 0))],
        out_specs=pl.BlockSpec((1, 1, _N, _P * _C), lambda l, b: (l, b, 0, 0)),
        out_shape=jax.ShapeDtypeStruct((4, _B, _N, _P * _C), F32),
        compiler_params=pltpu.CompilerParams(
            dimension_semantics=("parallel", "parallel")),
    )(xt, wt, bt)
    projs = yo.reshape(4, _B, _N, _P, _C).transpose(0, 1, 4, 2, 3)
    p1, p2, p3, p4 = projs[0], projs[1], projs[2], projs[3]

    w64 = jnp.concatenate([conv1_w[:, :, 0, :], conv2_w[:, :, 0, :]],
                          axis=0)
    w_all = w64.transpose(2, 0, 1).reshape(_KW * 64, _G * _C)
    bc = jnp.concatenate([conv1_b, conv2_b]).reshape(1, 1, 64)
    rw = res_w[:, :, 0, 0]
    rb = res_b.reshape(1, 1, _C)
    sw = skip_w[:, :, 0, 0]
    sb = skip_b.reshape(1, 1, _C)
    r4 = lambda a, t: a.reshape(_B, _C, _N, t)
    ins = [x, r4(x1, _T), r4(ltm, _TE), r4(lts, _TE),
           r4(x2, _T), r4(sem, _TE), r4(ses, _TE),
           r4(x3, _T), r4(stm, _TE), r4(sts, _TE),
           r4(x4, _T), r4(snm, _TE), r4(sns, _TE), p1, p2, p3, p4]
    blk = lambda t: pl.BlockSpec((1, _C, _NB, t), lambda b, n: (b, 0, n, 0))
    wfull = lambda *s: pl.BlockSpec(s, lambda b, n: (0,) * len(s))
    xz, s = pl.pallas_call(
        _conv_kernel,
        grid=(_B, _N // _NB),
        in_specs=[blk(_T), blk(_T), blk(_TE), blk(_TE),
                  blk(_T), blk(_TE), blk(_TE),
                  blk(_T), blk(_TE), blk(_TE),
                  blk(_T), blk(_TE), blk(_TE),
                  blk(_P), blk(_P), blk(_P), blk(_P),
                  wfull(_KW * 64, _G * _C), wfull(1, 1, 64),
                  wfull(_C, _C), wfull(1, 1, _C),
                  wfull(_C, _C), wfull(1, 1, _C)],
        out_specs=[blk(_TO - _P), blk(_P)],
        out_shape=[jax.ShapeDtypeStruct((_B, _C, _N, _TO - _P), F32),
                   jax.ShapeDtypeStruct((_B, _C, _N, _P), F32)],
        compiler_params=pltpu.CompilerParams(
            dimension_semantics=("parallel", "parallel"),
            vmem_limit_bytes=50 * 1024 * 1024),
    )(*ins, w_all, bc, rw, rb, sw, sb)

    tw = _KW + _P - 1
    sl = lambda a: a.reshape(_B, _C, _N, _TE)[..., _TE - tw:]
    aslab = jnp.concatenate(
        [sl(ltm), sl(lts), sl(sem), sl(ses), sl(stm), sl(sts),
         sl(snm), sl(sns)], axis=1).transpose(0, 1, 3, 2)
    sel = np.concatenate(
        [np.arange(g * _C, (g + 1) * _C) for g in (2, 3, 5, 6, 8, 9, 11, 12)])
    wa = w64[:, sel, :].transpose(2, 0, 1).reshape(_KW * 64, 8 * _C)
    saux = pl.pallas_call(
        _aux_kernel,
        grid=(_B,),
        in_specs=[pl.BlockSpec((1, 8 * _C, tw, _N), lambda b: (b, 0, 0, 0)),
                  pl.BlockSpec((_KW * 64, 8 * _C), lambda b: (0, 0)),
                  pl.BlockSpec((1, 1, 64), lambda b: (0, 0, 0)),
                  pl.BlockSpec((_C, _C), lambda b: (0, 0)),
                  pl.BlockSpec((1, 1, _C), lambda b: (0, 0, 0))],
        out_specs=pl.BlockSpec((1, _C, _P, _N), lambda b: (b, 0, 0, 0)),
        out_shape=jax.ShapeDtypeStruct((_B, _C, _P, _N), F32),
        compiler_params=pltpu.CompilerParams(
            dimension_semantics=("parallel",)),
    )(aslab, wa, bc, sw, sb)
    saux = saux.transpose(0, 1, 3, 2)

    return xz, saux, s


# speedup vs baseline: 3.7465x; 1.1486x over previous
import jax
import jax.numpy as jnp
import numpy as np
from jax.experimental import pallas as pl
from jax.experimental.pallas import tpu as pltpu

F32 = jnp.float32
_B, _C, _N, _T = 4, 32, 256, 288
_P = 12
_KW = 12
_PERIOD, _SHORT, _LONG = 24, 12, 48
_RK = 5
_TE = _T + _P
_TO = _TE + 1 - _KW + 1
_NB = 8
_G = 13


def _win_matrix(L):
    s = np.arange(_T)[:, None]
    t = np.maximum(np.arange(_T)[None, :], L - 1)
    return (((s <= t) & (s >= t - L + 1)).astype(np.float32) / L)


def _season_matrix(P):
    s = np.arange(_T)[:, None]
    t = np.arange(_T)[None, :]
    return ((s % P) == (t % P)).astype(np.float32) * (P / _T)


_M48 = _win_matrix(_LONG)
_S24 = _season_matrix(_PERIOD)
_M12 = _win_matrix(_SHORT)


def _adj_kernel(emb_ref, adj_ref):
    e = emb_ref[...]
    logits = jax.lax.dot_general(e, e, (((1,), (1,)), ((), ())),
                                 preferred_element_type=F32)
    r = jax.lax.broadcasted_iota(jnp.int32, (_N, _N), 0)
    c = jax.lax.broadcasted_iota(jnp.int32, (_N, _N), 1)
    logits = logits - jnp.where(r == c, 10.0, 0.0).astype(F32)
    logits = logits - jnp.max(logits, axis=-1, keepdims=True)
    p = jnp.exp(logits)
    adj_ref[...] = p / jnp.sum(p, axis=-1, keepdims=True)


def _chain_kernel(x_ref, m48_ref, s24_ref, m12_ref, adj_ref,
                  x1_ref, ltm_ref, lts_ref,
                  x2_ref, sem_ref, ses_ref,
                  x3_ref, stm_ref, sts_ref,
                  x4_ref, snm_ref, sns_ref,
                  tltm_ref, tlts_ref, tsem_ref, tses_ref,
                  tstm_ref, tsts_ref, tsnm_ref, tsns_ref):
    xb = x_ref[0]

    def tnorm(v, M):
        m = jnp.dot(v, M, preferred_element_type=F32)
        q = jnp.dot(v * v, M, preferred_element_type=F32)
        var = q - m * m + 1e-5
        out = (v - m) * jax.lax.rsqrt(var + 1.0)
        return out, m, jnp.sqrt(var)

    x1, m1, s1 = tnorm(xb, m48_ref[...])
    x2, m2, s2 = tnorm(x1, s24_ref[...])
    x3, m3, s3 = tnorm(x2, m12_ref[...])

    a = adj_ref[...]
    ms = jnp.dot(a, x3, preferred_element_type=F32)
    qs = jnp.dot(a, x3 * x3, preferred_element_type=F32)
    vs = qs - ms * ms + 1e-5
    x4 = (x3 - ms) * jax.lax.rsqrt(vs + 1.0)
    ss = jnp.sqrt(vs)

    def extc(m):
        return jnp.concatenate(
            [m, jnp.broadcast_to(m[:, _T - 1:_T], (_N, _P))], axis=-1)

    def exts(m):
        return jnp.concatenate([m, m[:, :_P]], axis=-1)

    x1_ref[0] = x1
    x2_ref[0] = x2
    x3_ref[0] = x3
    x4_ref[0] = x4
    tw0 = _TE - (_KW + _P - 1)
    for eref, tref, stat in ((ltm_ref, tltm_ref, extc(m1)),
                             (lts_ref, tlts_ref, extc(s1)),
                             (sem_ref, tsem_ref, exts(m2)),
                             (ses_ref, tses_ref, exts(s2)),
                             (stm_ref, tstm_ref, extc(m3)),
                             (sts_ref, tsts_ref, extc(s3)),
                             (snm_ref, tsnm_ref, extc(ms)),
                             (sns_ref, tsns_ref, extc(ss))):
        eref[0] = stat
        tref[0] = stat[:, tw0:].T


def _rex_kernel(x_ref, w_ref, b_ref, o_ref):
    o_ref[0, 0] = (jnp.dot(x_ref[0, 0], w_ref[0],
                           preferred_element_type=F32) + b_ref[0, 0])


def _conv_kernel(x_ref, x1_ref, ltm_ref, lts_ref, x2_ref, sem_ref, ses_ref,
                 x3_ref, stm_ref, sts_ref, x4_ref, snm_ref, sns_ref,
                 p1_ref, p2_ref, p3_ref, p4_ref,
                 w_ref, bc_ref, rw_ref, rb_ref, sw_ref, sb_ref,
                 xz_ref, s_ref):
    def body(ref):
        b = ref[0]
        return jnp.concatenate(
            [b, jnp.broadcast_to(b[:, :, _T - 1:_T], (_C, _NB, _P))], axis=-1)

    def withp(ref, pref):
        return jnp.concatenate([ref[0], pref[0]], axis=-1)

    groups = [body(x_ref), withp(x1_ref, p1_ref), ltm_ref[0], lts_ref[0],
              withp(x2_ref, p2_ref), sem_ref[0], ses_ref[0],
              withp(x3_ref, p3_ref), stm_ref[0], sts_ref[0],
              withp(x4_ref, p4_ref), snm_ref[0], sns_ref[0]]
    slab = jnp.concatenate(groups, axis=0)
    slab = jnp.concatenate(
        [jnp.zeros((_G * _C, _NB, 1), F32), slab], axis=-1)

    y = jnp.einsum('oc,cnt->ont', w_ref[...], slab,
                   preferred_element_type=F32)
    acc = jnp.zeros((64, _NB, _TO), F32)
    for k in range(_KW):
        acc = acc + y[64 * k:64 * (k + 1), :, k:k + _TO]
    acc = acc + bc_ref[0, 0][:, None, None]

    prod = acc[:_C] * acc[_C:]
    xz = jnp.einsum('oc,cnt->ont', rw_ref[...], prod[:, :, :_TO - _P],
                    preferred_element_type=F32) + rb_ref[0, 0][:, None, None]
    sk = jnp.einsum('oc,cnt->ont', sw_ref[...], prod[:, :, _TO - _P:],
                    preferred_element_type=F32) + sb_ref[0, 0][:, None, None]
    xz_ref[0] = xz
    s_ref[0] = sk


def _aux_kernel(t1_ref, t2_ref, t3_ref, t4_ref, t5_ref, t6_ref, t7_ref,
                t8_ref, w_ref, bc_ref, sw_ref, sb_ref, o_ref):
    slab = jnp.concatenate(
        [r[0] for r in (t1_ref, t2_ref, t3_ref, t4_ref,
                        t5_ref, t6_ref, t7_ref, t8_ref)], axis=0)
    y = jnp.einsum('oc,ctn->otn', w_ref[...], slab,
                   preferred_element_type=F32)
    acc = jnp.zeros((64, _P, _N), F32)
    for k in range(_KW):
        acc = acc + y[64 * k:64 * (k + 1), k:k + _P, :]
    acc = acc + bc_ref[0, 0][:, None, None]
    prod = acc[:_C] * acc[_C:]
    res = jnp.einsum('oc,ctn->otn', sw_ref[...], prod,
                     preferred_element_type=F32) + sb_ref[0, 0][:, None, None]
    o_ref[0] = res.transpose(0, 2, 1)


def kernel(x, node_embedding, conv1_w, conv1_b, conv2_w, conv2_b,
           skip_w, skip_b, res_w, res_b,
           re1_w, re1_b, re2_w, re2_b, re3_w, re3_b, re4_w, re4_b):
    adj = pl.pallas_call(
        _adj_kernel,
        out_shape=jax.ShapeDtypeStruct((_N, _N), F32),
    )(node_embedding)

    xf = x.reshape(_B * _C, _N, _T)
    big = jax.ShapeDtypeStruct((_B * _C, _N, _T), F32)
    ext = jax.ShapeDtypeStruct((_B * _C, _N, _TE), F32)
    row = lambda t: pl.BlockSpec((1, _N, t), lambda i: (i, 0, 0))
    full2 = lambda a, b: pl.BlockSpec((a, b), lambda i: (0, 0))
    tw = _KW + _P - 1
    tl = jax.ShapeDtypeStruct((_B * _C, tw, _N), F32)
    tspec = pl.BlockSpec((1, tw, _N), lambda i: (i, 0, 0))
    outs = pl.pallas_call(
        _chain_kernel,
        grid=(_B * _C,),
        in_specs=[row(_T), full2(_T, _T), full2(_T, _T), full2(_T, _T),
                  full2(_N, _N)],
        out_specs=[row(_T), row(_TE), row(_TE),
                   row(_T), row(_TE), row(_TE),
                   row(_T), row(_TE), row(_TE),
                   row(_T), row(_TE), row(_TE)] + [tspec] * 8,
        out_shape=[big, ext, ext, big, ext, ext, big, ext, ext,
                   big, ext, ext] + [tl] * 8,
        compiler_params=pltpu.CompilerParams(
            dimension_semantics=("parallel",)),
    )(xf, jnp.asarray(_M48), jnp.asarray(_S24), jnp.asarray(_M12), adj)
    (x1, ltm, lts, x2, sem, ses, x3, stm, sts, x4, snm, sns,
     tltm, tlts, tsem, tses, tstm, tsts, tsnm, tsns) = outs

    tails = jnp.stack([x1, x2, x3, x4]).reshape(
        4, _B, _C, _N, _T)[..., _T - _RK:]
    xt = tails.transpose(0, 1, 3, 2, 4).reshape(4, _B, _N, _C * _RK)
    wt = jnp.stack([re1_w, re2_w, re3_w, re4_w])[:, :, :, 0, :].transpose(
        0, 2, 3, 1).reshape(4, _C * _RK, _P * _C)
    bt = jnp.stack([re1_b, re2_b, re3_b, re4_b]).reshape(4, 1, _P * _C)
    yo = pl.pallas_call(
        _rex_kernel,
        grid=(4, _B),
        in_specs=[pl.BlockSpec((1, 1, _N, _C * _RK), lambda l, b: (l, b, 0, 0)),
                  pl.BlockSpec((1, _C * _RK, _P * _C), lambda l, b: (l, 0, 0)),
                  pl.BlockSpec((1, 1, _P * _C), lambda l, b: (l, 0, 0))],
        out_specs=pl.BlockSpec((1, 1, _N, _P * _C), lambda l, b: (l, b, 0, 0)),
        out_shape=jax.ShapeDtypeStruct((4, _B, _N, _P * _C), F32),
        compiler_params=pltpu.CompilerParams(
            dimension_semantics=("parallel", "parallel")),
    )(xt, wt, bt)
    projs = yo.reshape(4, _B, _N, _P, _C).transpose(0, 1, 4, 2, 3)
    p1, p2, p3, p4 = projs[0], projs[1], projs[2], projs[3]

    w64 = jnp.concatenate([conv1_w[:, :, 0, :], conv2_w[:, :, 0, :]],
                          axis=0)
    w_all = w64.transpose(2, 0, 1).reshape(_KW * 64, _G * _C)
    bc = jnp.concatenate([conv1_b, conv2_b]).reshape(1, 1, 64)
    rw = res_w[:, :, 0, 0]
    rb = res_b.reshape(1, 1, _C)
    sw = skip_w[:, :, 0, 0]
    sb = skip_b.reshape(1, 1, _C)
    r4 = lambda a, t: a.reshape(_B, _C, _N, t)
    ins = [x, r4(x1, _T), r4(ltm, _TE), r4(lts, _TE),
           r4(x2, _T), r4(sem, _TE), r4(ses, _TE),
           r4(x3, _T), r4(stm, _TE), r4(sts, _TE),
           r4(x4, _T), r4(snm, _TE), r4(sns, _TE), p1, p2, p3, p4]
    blk = lambda t: pl.BlockSpec((1, _C, _NB, t), lambda b, n: (b, 0, n, 0))
    wfull = lambda *s: pl.BlockSpec(s, lambda b, n: (0,) * len(s))
    xz, s = pl.pallas_call(
        _conv_kernel,
        grid=(_B, _N // _NB),
        in_specs=[blk(_T), blk(_T), blk(_TE), blk(_TE),
                  blk(_T), blk(_TE), blk(_TE),
                  blk(_T), blk(_TE), blk(_TE),
                  blk(_T), blk(_TE), blk(_TE),
                  blk(_P), blk(_P), blk(_P), blk(_P),
                  wfull(_KW * 64, _G * _C), wfull(1, 1, 64),
                  wfull(_C, _C), wfull(1, 1, _C),
                  wfull(_C, _C), wfull(1, 1, _C)],
        out_specs=[blk(_TO - _P), blk(_P)],
        out_shape=[jax.ShapeDtypeStruct((_B, _C, _N, _TO - _P), F32),
                   jax.ShapeDtypeStruct((_B, _C, _N, _P), F32)],
        compiler_params=pltpu.CompilerParams(
            dimension_semantics=("parallel", "parallel"),
            vmem_limit_bytes=50 * 1024 * 1024),
    )(*ins, w_all, bc, rw, rb, sw, sb)

    sel = np.concatenate(
        [np.arange(g * _C, (g + 1) * _C) for g in (2, 3, 5, 6, 8, 9, 11, 12)])
    wa = w64[:, sel, :].transpose(2, 0, 1).reshape(_KW * 64, 8 * _C)
    rt = lambda a: a.reshape(_B, _C, tw, _N)
    ttspec = pl.BlockSpec((1, _C, tw, _N), lambda b: (b, 0, 0, 0))
    saux = pl.pallas_call(
        _aux_kernel,
        grid=(_B,),
        in_specs=[ttspec] * 8 +
                 [pl.BlockSpec((_KW * 64, 8 * _C), lambda b: (0, 0)),
                  pl.BlockSpec((1, 1, 64), lambda b: (0, 0, 0)),
                  pl.BlockSpec((_C, _C), lambda b: (0, 0)),
                  pl.BlockSpec((1, 1, _C), lambda b: (0, 0, 0))],
        out_specs=pl.BlockSpec((1, _C, _N, _P), lambda b: (b, 0, 0, 0)),
        out_shape=jax.ShapeDtypeStruct((_B, _C, _N, _P), F32),
        compiler_params=pltpu.CompilerParams(
            dimension_semantics=("parallel",)),
    )(rt(tltm), rt(tlts), rt(tsem), rt(tses), rt(tstm), rt(tsts),
      rt(tsnm), rt(tsns), wa, bc, sw, sb)

    return xz, saux, s


# speedup vs baseline: 4.0035x; 1.0686x over previous
import jax
import jax.numpy as jnp
import numpy as np
from jax.experimental import pallas as pl
from jax.experimental.pallas import tpu as pltpu

F32 = jnp.float32
_B, _C, _N, _T = 4, 32, 256, 288
_P = 12
_KW = 12
_PERIOD, _SHORT, _LONG = 24, 12, 48
_RK = 5
_TE = _T + _P
_TO = _TE + 1 - _KW + 1
_NB = 8
_G = 13


def _win_matrix(L):
    s = np.arange(_T)[:, None]
    t = np.maximum(np.arange(_T)[None, :], L - 1)
    return (((s <= t) & (s >= t - L + 1)).astype(np.float32) / L)


def _season_matrix(P):
    s = np.arange(_T)[:, None]
    t = np.arange(_T)[None, :]
    return ((s % P) == (t % P)).astype(np.float32) * (P / _T)


_M48 = _win_matrix(_LONG)
_S24 = _season_matrix(_PERIOD)
_M12 = _win_matrix(_SHORT)


def _adj_kernel(emb_ref, adj_ref):
    e = emb_ref[...]
    logits = jax.lax.dot_general(e, e, (((1,), (1,)), ((), ())),
                                 preferred_element_type=F32)
    r = jax.lax.broadcasted_iota(jnp.int32, (_N, _N), 0)
    c = jax.lax.broadcasted_iota(jnp.int32, (_N, _N), 1)
    logits = logits - jnp.where(r == c, 10.0, 0.0).astype(F32)
    logits = logits - jnp.max(logits, axis=-1, keepdims=True)
    p = jnp.exp(logits)
    adj_ref[...] = p / jnp.sum(p, axis=-1, keepdims=True)


def _chain_kernel(x_ref, m48_ref, s24_ref, m12_ref, adj_ref,
                  x1_ref, ltm_ref, lts_ref,
                  x2_ref, sem_ref, ses_ref,
                  x3_ref, stm_ref, sts_ref,
                  x4_ref, snm_ref, sns_ref,
                  tltm_ref, tlts_ref, tsem_ref, tses_ref,
                  tstm_ref, tsts_ref, tsnm_ref, tsns_ref):
    xb = x_ref[0]

    def tnorm(v, M):
        m = jnp.dot(v, M, preferred_element_type=F32)
        q = jnp.dot(v * v, M, preferred_element_type=F32)
        var = q - m * m + 1e-5
        out = (v - m) * jax.lax.rsqrt(var + 1.0)
        return out, m, jnp.sqrt(var)

    x1, m1, s1 = tnorm(xb, m48_ref[...])
    x2, m2, s2 = tnorm(x1, s24_ref[...])
    x3, m3, s3 = tnorm(x2, m12_ref[...])

    a = adj_ref[...]
    ms = jnp.dot(a, x3, preferred_element_type=F32)
    qs = jnp.dot(a, x3 * x3, preferred_element_type=F32)
    vs = qs - ms * ms + 1e-5
    x4 = (x3 - ms) * jax.lax.rsqrt(vs + 1.0)
    ss = jnp.sqrt(vs)

    def extc(m):
        return jnp.concatenate(
            [m, jnp.broadcast_to(m[:, _T - 1:_T], (_N, _P))], axis=-1)

    def exts(m):
        return jnp.concatenate([m, m[:, :_P]], axis=-1)

    x1_ref[0] = x1
    x2_ref[0] = x2
    x3_ref[0] = x3
    x4_ref[0] = x4
    tw0 = _TE - (_KW + _P - 1)
    for eref, tref, stat in ((ltm_ref, tltm_ref, extc(m1)),
                             (lts_ref, tlts_ref, extc(s1)),
                             (sem_ref, tsem_ref, exts(m2)),
                             (ses_ref, tses_ref, exts(s2)),
                             (stm_ref, tstm_ref, extc(m3)),
                             (sts_ref, tsts_ref, extc(s3)),
                             (snm_ref, tsnm_ref, extc(ms)),
                             (sns_ref, tsns_ref, extc(ss))):
        eref[0] = stat
        tref[0] = stat[:, tw0:].T


def _rex_kernel(x_ref, w_ref, b_ref, o_ref):
    o_ref[0, 0] = (jnp.dot(x_ref[0, 0], w_ref[0],
                           preferred_element_type=F32) + b_ref[0, 0])


def _conv_kernel(x_ref, x1_ref, ltm_ref, lts_ref, x2_ref, sem_ref, ses_ref,
                 x3_ref, stm_ref, sts_ref, x4_ref, snm_ref, sns_ref,
                 p1_ref, p2_ref, p3_ref, p4_ref,
                 w_ref, bc_ref, rw_ref, rb_ref, sw_ref, sb_ref,
                 xz_ref, s_ref):
    def body(ref):
        b = ref[0]
        return jnp.concatenate(
            [b, jnp.broadcast_to(b[:, :, _T - 1:_T], (_C, _NB, _P))], axis=-1)

    def withp(ref, pref):
        return jnp.concatenate([ref[0], pref[0]], axis=-1)

    groups = [body(x_ref), withp(x1_ref, p1_ref), ltm_ref[0], lts_ref[0],
              withp(x2_ref, p2_ref), sem_ref[0], ses_ref[0],
              withp(x3_ref, p3_ref), stm_ref[0], sts_ref[0],
              withp(x4_ref, p4_ref), snm_ref[0], sns_ref[0]]
    slab = jnp.concatenate(groups, axis=0)
    slab = jnp.concatenate(
        [jnp.zeros((_G * _C, _NB, 1), F32), slab], axis=-1)

    y = jnp.einsum('oc,cnt->ont', w_ref[...].astype(jnp.bfloat16),
                   slab.astype(jnp.bfloat16),
                   preferred_element_type=F32)
    acc = jnp.zeros((64, _NB, _TO), F32)
    for k in range(_KW):
        acc = acc + y[64 * k:64 * (k + 1), :, k:k + _TO]
    acc = acc + bc_ref[0, 0][:, None, None]

    prod = acc[:_C] * acc[_C:]
    xz = jnp.einsum('oc,cnt->ont', rw_ref[...], prod[:, :, :_TO - _P],
                    preferred_element_type=F32) + rb_ref[0, 0][:, None, None]
    sk = jnp.einsum('oc,cnt->ont', sw_ref[...], prod[:, :, _TO - _P:],
                    preferred_element_type=F32) + sb_ref[0, 0][:, None, None]
    xz_ref[0] = xz
    s_ref[0] = sk


def _aux_kernel(t1_ref, t2_ref, t3_ref, t4_ref, t5_ref, t6_ref, t7_ref,
                t8_ref, w_ref, bc_ref, sw_ref, sb_ref, o_ref):
    slab = jnp.concatenate(
        [r[0] for r in (t1_ref, t2_ref, t3_ref, t4_ref,
                        t5_ref, t6_ref, t7_ref, t8_ref)], axis=0)
    y = jnp.einsum('oc,ctn->otn', w_ref[...], slab,
                   preferred_element_type=F32)
    acc = jnp.zeros((64, _P, _N), F32)
    for k in range(_KW):
        acc = acc + y[64 * k:64 * (k + 1), k:k + _P, :]
    acc = acc + bc_ref[0, 0][:, None, None]
    prod = acc[:_C] * acc[_C:]
    res = jnp.einsum('oc,ctn->otn', sw_ref[...], prod,
                     preferred_element_type=F32) + sb_ref[0, 0][:, None, None]
    o_ref[0] = res.transpose(0, 2, 1)


def kernel(x, node_embedding, conv1_w, conv1_b, conv2_w, conv2_b,
           skip_w, skip_b, res_w, res_b,
           re1_w, re1_b, re2_w, re2_b, re3_w, re3_b, re4_w, re4_b):
    adj = pl.pallas_call(
        _adj_kernel,
        out_shape=jax.ShapeDtypeStruct((_N, _N), F32),
    )(node_embedding)

    xf = x.reshape(_B * _C, _N, _T)
    big = jax.ShapeDtypeStruct((_B * _C, _N, _T), F32)
    ext = jax.ShapeDtypeStruct((_B * _C, _N, _TE), F32)
    row = lambda t: pl.BlockSpec((1, _N, t), lambda i: (i, 0, 0))
    full2 = lambda a, b: pl.BlockSpec((a, b), lambda i: (0, 0))
    tw = _KW + _P - 1
    tl = jax.ShapeDtypeStruct((_B * _C, tw, _N), F32)
    tspec = pl.BlockSpec((1, tw, _N), lambda i: (i, 0, 0))
    outs = pl.pallas_call(
        _chain_kernel,
        grid=(_B * _C,),
        in_specs=[row(_T), full2(_T, _T), full2(_T, _T), full2(_T, _T),
                  full2(_N, _N)],
        out_specs=[row(_T), row(_TE), row(_TE),
                   row(_T), row(_TE), row(_TE),
                   row(_T), row(_TE), row(_TE),
                   row(_T), row(_TE), row(_TE)] + [tspec] * 8,
        out_shape=[big, ext, ext, big, ext, ext, big, ext, ext,
                   big, ext, ext] + [tl] * 8,
        compiler_params=pltpu.CompilerParams(
            dimension_semantics=("parallel",)),
    )(xf, jnp.asarray(_M48), jnp.asarray(_S24), jnp.asarray(_M12), adj)
    (x1, ltm, lts, x2, sem, ses, x3, stm, sts, x4, snm, sns,
     tltm, tlts, tsem, tses, tstm, tsts, tsnm, tsns) = outs

    tails = jnp.stack([x1, x2, x3, x4]).reshape(
        4, _B, _C, _N, _T)[..., _T - _RK:]
    xt = tails.transpose(0, 1, 3, 2, 4).reshape(4, _B, _N, _C * _RK)
    wt = jnp.stack([re1_w, re2_w, re3_w, re4_w])[:, :, :, 0, :].transpose(
        0, 2, 3, 1).reshape(4, _C * _RK, _P * _C)
    bt = jnp.stack([re1_b, re2_b, re3_b, re4_b]).reshape(4, 1, _P * _C)
    yo = pl.pallas_call(
        _rex_kernel,
        grid=(4, _B),
        in_specs=[pl.BlockSpec((1, 1, _N, _C * _RK), lambda l, b: (l, b, 0, 0)),
                  pl.BlockSpec((1, _C * _RK, _P * _C), lambda l, b: (l, 0, 0)),
                  pl.BlockSpec((1, 1, _P * _C), lambda l, b: (l, 0, 0))],
        out_specs=pl.BlockSpec((1, 1, _N, _P * _C), lambda l, b: (l, b, 0, 0)),
        out_shape=jax.ShapeDtypeStruct((4, _B, _N, _P * _C), F32),
        compiler_params=pltpu.CompilerParams(
            dimension_semantics=("parallel", "parallel")),
    )(xt, wt, bt)
    projs = yo.reshape(4, _B, _N, _P, _C).transpose(0, 1, 4, 2, 3)
    p1, p2, p3, p4 = projs[0], projs[1], projs[2], projs[3]

    w64 = jnp.concatenate([conv1_w[:, :, 0, :], conv2_w[:, :, 0, :]],
                          axis=0)
    w_all = w64.transpose(2, 0, 1).reshape(_KW * 64, _G * _C)
    bc = jnp.concatenate([conv1_b, conv2_b]).reshape(1, 1, 64)
    rw = res_w[:, :, 0, 0]
    rb = res_b.reshape(1, 1, _C)
    sw = skip_w[:, :, 0, 0]
    sb = skip_b.reshape(1, 1, _C)
    r4 = lambda a, t: a.reshape(_B, _C, _N, t)
    ins = [x, r4(x1, _T), r4(ltm, _TE), r4(lts, _TE),
           r4(x2, _T), r4(sem, _TE), r4(ses, _TE),
           r4(x3, _T), r4(stm, _TE), r4(sts, _TE),
           r4(x4, _T), r4(snm, _TE), r4(sns, _TE), p1, p2, p3, p4]
    blk = lambda t: pl.BlockSpec((1, _C, _NB, t), lambda b, n: (b, 0, n, 0))
    wfull = lambda *s: pl.BlockSpec(s, lambda b, n: (0,) * len(s))
    xz, s = pl.pallas_call(
        _conv_kernel,
        grid=(_B, _N // _NB),
        in_specs=[blk(_T), blk(_T), blk(_TE), blk(_TE),
                  blk(_T), blk(_TE), blk(_TE),
                  blk(_T), blk(_TE), blk(_TE),
                  blk(_T), blk(_TE), blk(_TE),
                  blk(_P), blk(_P), blk(_P), blk(_P),
                  wfull(_KW * 64, _G * _C), wfull(1, 1, 64),
                  wfull(_C, _C), wfull(1, 1, _C),
                  wfull(_C, _C), wfull(1, 1, _C)],
        out_specs=[blk(_TO - _P), blk(_P)],
        out_shape=[jax.ShapeDtypeStruct((_B, _C, _N, _TO - _P), F32),
                   jax.ShapeDtypeStruct((_B, _C, _N, _P), F32)],
        compiler_params=pltpu.CompilerParams(
            dimension_semantics=("parallel", "parallel"),
            vmem_limit_bytes=50 * 1024 * 1024),
    )(*ins, w_all, bc, rw, rb, sw, sb)

    sel = np.concatenate(
        [np.arange(g * _C, (g + 1) * _C) for g in (2, 3, 5, 6, 8, 9, 11, 12)])
    wa = w64[:, sel, :].transpose(2, 0, 1).reshape(_KW * 64, 8 * _C)
    rt = lambda a: a.reshape(_B, _C, tw, _N)
    ttspec = pl.BlockSpec((1, _C, tw, _N), lambda b: (b, 0, 0, 0))
    saux = pl.pallas_call(
        _aux_kernel,
        grid=(_B,),
        in_specs=[ttspec] * 8 +
                 [pl.BlockSpec((_KW * 64, 8 * _C), lambda b: (0, 0)),
                  pl.BlockSpec((1, 1, 64), lambda b: (0, 0, 0)),
                  pl.BlockSpec((_C, _C), lambda b: (0, 0)),
                  pl.BlockSpec((1, 1, _C), lambda b: (0, 0, 0))],
        out_specs=pl.BlockSpec((1, _C, _N, _P), lambda b: (b, 0, 0, 0)),
        out_shape=jax.ShapeDtypeStruct((_B, _C, _N, _P), F32),
        compiler_params=pltpu.CompilerParams(
            dimension_semantics=("parallel",)),
    )(rt(tltm), rt(tlts), rt(tsem), rt(tses), rt(tstm), rt(tsts),
      rt(tsnm), rt(tsns), wa, bc, sw, sb)

    return xz, saux, s


# speedup vs baseline: 4.2170x; 1.0533x over previous
import jax
import jax.numpy as jnp
import numpy as np
from jax.experimental import pallas as pl
from jax.experimental.pallas import tpu as pltpu

F32 = jnp.float32
_B, _C, _N, _T = 4, 32, 256, 288
_P = 12
_KW = 12
_PERIOD, _SHORT, _LONG = 24, 12, 48
_RK = 5
_TE = _T + _P
_TO = _TE + 1 - _KW + 1
_NB = 16
BF16 = jnp.bfloat16
_G = 13


def _win_matrix(L):
    s = np.arange(_T)[:, None]
    t = np.maximum(np.arange(_T)[None, :], L - 1)
    return (((s <= t) & (s >= t - L + 1)).astype(np.float32) / L)


def _season_matrix(P):
    s = np.arange(_T)[:, None]
    t = np.arange(_T)[None, :]
    return ((s % P) == (t % P)).astype(np.float32) * (P / _T)


_M48 = _win_matrix(_LONG)
_S24 = _season_matrix(_PERIOD)
_M12 = _win_matrix(_SHORT)


def _adj_kernel(emb_ref, adj_ref):
    e = emb_ref[...]
    logits = jax.lax.dot_general(e, e, (((1,), (1,)), ((), ())),
                                 preferred_element_type=F32)
    r = jax.lax.broadcasted_iota(jnp.int32, (_N, _N), 0)
    c = jax.lax.broadcasted_iota(jnp.int32, (_N, _N), 1)
    logits = logits - jnp.where(r == c, 10.0, 0.0).astype(F32)
    logits = logits - jnp.max(logits, axis=-1, keepdims=True)
    p = jnp.exp(logits)
    adj_ref[...] = p / jnp.sum(p, axis=-1, keepdims=True)


def _chain_kernel(x_ref, m48_ref, s24_ref, m12_ref, adj_ref,
                  x1_ref, ltm_ref, lts_ref,
                  x2_ref, sem_ref, ses_ref,
                  x3_ref, stm_ref, sts_ref,
                  x4_ref, snm_ref, sns_ref,
                  tltm_ref, tlts_ref, tsem_ref, tses_ref,
                  tstm_ref, tsts_ref, tsnm_ref, tsns_ref):
    xb = x_ref[0]

    def tnorm(v, M):
        m = jnp.dot(v, M, preferred_element_type=F32)
        q = jnp.dot(v * v, M, preferred_element_type=F32)
        var = q - m * m + 1e-5
        out = (v - m) * jax.lax.rsqrt(var + 1.0)
        return out, m, jnp.sqrt(var)

    x1, m1, s1 = tnorm(xb, m48_ref[...])
    x2, m2, s2 = tnorm(x1, s24_ref[...])
    x3, m3, s3 = tnorm(x2, m12_ref[...])

    a = adj_ref[...]
    ms = jnp.dot(a, x3, preferred_element_type=F32)
    qs = jnp.dot(a, x3 * x3, preferred_element_type=F32)
    vs = qs - ms * ms + 1e-5
    x4 = (x3 - ms) * jax.lax.rsqrt(vs + 1.0)
    ss = jnp.sqrt(vs)

    def extc(m):
        return jnp.concatenate(
            [m, jnp.broadcast_to(m[:, _T - 1:_T], (_N, _P))], axis=-1)

    def exts(m):
        return jnp.concatenate([m, m[:, :_P]], axis=-1)

    x1_ref[0] = x1.astype(BF16)
    x2_ref[0] = x2.astype(BF16)
    x3_ref[0] = x3.astype(BF16)
    x4_ref[0] = x4.astype(BF16)
    tw0 = _TE - (_KW + _P - 1)
    for eref, tref, stat in ((ltm_ref, tltm_ref, extc(m1)),
                             (lts_ref, tlts_ref, extc(s1)),
                             (sem_ref, tsem_ref, exts(m2)),
                             (ses_ref, tses_ref, exts(s2)),
                             (stm_ref, tstm_ref, extc(m3)),
                             (sts_ref, tsts_ref, extc(s3)),
                             (snm_ref, tsnm_ref, extc(ms)),
                             (sns_ref, tsns_ref, extc(ss))):
        eref[0] = stat.astype(BF16)
        tref[0] = stat[:, tw0:].T


def _rex_kernel(x_ref, w_ref, b_ref, o_ref):
    o_ref[0, 0] = (jnp.dot(x_ref[0, 0], w_ref[0],
                           preferred_element_type=F32)
                   + b_ref[0, 0]).astype(BF16)


def _conv_kernel(x_ref, x1_ref, ltm_ref, lts_ref, x2_ref, sem_ref, ses_ref,
                 x3_ref, stm_ref, sts_ref, x4_ref, snm_ref, sns_ref,
                 p1_ref, p2_ref, p3_ref, p4_ref,
                 w_ref, bc_ref, rw_ref, rb_ref, sw_ref, sb_ref,
                 xz_ref, s_ref):
    def body(ref):
        b = ref[0]
        return jnp.concatenate(
            [b, jnp.broadcast_to(b[:, :, _T - 1:_T], (_C, _NB, _P))], axis=-1)

    def withp(ref, pref):
        return jnp.concatenate([ref[0], pref[0]], axis=-1)

    groups = [body(x_ref), withp(x1_ref, p1_ref), ltm_ref[0], lts_ref[0],
              withp(x2_ref, p2_ref), sem_ref[0], ses_ref[0],
              withp(x3_ref, p3_ref), stm_ref[0], sts_ref[0],
              withp(x4_ref, p4_ref), snm_ref[0], sns_ref[0]]
    slab = jnp.concatenate(groups, axis=0)
    slab = jnp.concatenate(
        [jnp.zeros((_G * _C, _NB, 1), BF16), slab], axis=-1)

    y = jnp.einsum('oc,cnt->ont', w_ref[...], slab,
                   preferred_element_type=F32)
    acc = jnp.zeros((64, _NB, _TO), F32)
    for k in range(_KW):
        acc = acc + y[64 * k:64 * (k + 1), :, k:k + _TO]
    acc = acc + bc_ref[0, 0][:, None, None]

    prod = acc[:_C] * acc[_C:]
    xz = jnp.einsum('oc,cnt->ont', rw_ref[...], prod[:, :, :_TO - _P],
                    preferred_element_type=F32) + rb_ref[0, 0][:, None, None]
    sk = jnp.einsum('oc,cnt->ont', sw_ref[...], prod[:, :, _TO - _P:],
                    preferred_element_type=F32) + sb_ref[0, 0][:, None, None]
    xz_ref[0] = xz
    s_ref[0] = sk


def _aux_kernel(t1_ref, t2_ref, t3_ref, t4_ref, t5_ref, t6_ref, t7_ref,
                t8_ref, w_ref, bc_ref, sw_ref, sb_ref, o_ref):
    slab = jnp.concatenate(
        [r[0] for r in (t1_ref, t2_ref, t3_ref, t4_ref,
                        t5_ref, t6_ref, t7_ref, t8_ref)], axis=0)
    y = jnp.einsum('oc,ctn->otn', w_ref[...], slab,
                   preferred_element_type=F32)
    acc = jnp.zeros((64, _P, _N), F32)
    for k in range(_KW):
        acc = acc + y[64 * k:64 * (k + 1), k:k + _P, :]
    acc = acc + bc_ref[0, 0][:, None, None]
    prod = acc[:_C] * acc[_C:]
    res = jnp.einsum('oc,ctn->otn', sw_ref[...], prod,
                     preferred_element_type=F32) + sb_ref[0, 0][:, None, None]
    o_ref[0] = res.transpose(0, 2, 1)


def kernel(x, node_embedding, conv1_w, conv1_b, conv2_w, conv2_b,
           skip_w, skip_b, res_w, res_b,
           re1_w, re1_b, re2_w, re2_b, re3_w, re3_b, re4_w, re4_b):
    adj = pl.pallas_call(
        _adj_kernel,
        out_shape=jax.ShapeDtypeStruct((_N, _N), F32),
    )(node_embedding)

    xf = x.reshape(_B * _C, _N, _T)
    big = jax.ShapeDtypeStruct((_B * _C, _N, _T), BF16)
    ext = jax.ShapeDtypeStruct((_B * _C, _N, _TE), BF16)
    row = lambda t: pl.BlockSpec((1, _N, t), lambda i: (i, 0, 0))
    full2 = lambda a, b: pl.BlockSpec((a, b), lambda i: (0, 0))
    tw = _KW + _P - 1
    tl = jax.ShapeDtypeStruct((_B * _C, tw, _N), F32)
    tspec = pl.BlockSpec((1, tw, _N), lambda i: (i, 0, 0))
    outs = pl.pallas_call(
        _chain_kernel,
        grid=(_B * _C,),
        in_specs=[row(_T), full2(_T, _T), full2(_T, _T), full2(_T, _T),
                  full2(_N, _N)],
        out_specs=[row(_T), row(_TE), row(_TE),
                   row(_T), row(_TE), row(_TE),
                   row(_T), row(_TE), row(_TE),
                   row(_T), row(_TE), row(_TE)] + [tspec] * 8,
        out_shape=[big, ext, ext, big, ext, ext, big, ext, ext,
                   big, ext, ext] + [tl] * 8,
        compiler_params=pltpu.CompilerParams(
            dimension_semantics=("parallel",)),
    )(xf, jnp.asarray(_M48), jnp.asarray(_S24), jnp.asarray(_M12), adj)
    (x1, ltm, lts, x2, sem, ses, x3, stm, sts, x4, snm, sns,
     tltm, tlts, tsem, tses, tstm, tsts, tsnm, tsns) = outs

    tails = jnp.stack([x1, x2, x3, x4]).reshape(
        4, _B, _C, _N, _T)[..., _T - _RK:]
    xt = tails.transpose(0, 1, 3, 2, 4).reshape(4, _B, _N, _C * _RK)
    wt = jnp.stack([re1_w, re2_w, re3_w, re4_w])[:, :, :, 0, :].transpose(
        0, 2, 3, 1).reshape(4, _C * _RK, _P * _C).astype(BF16)
    bt = jnp.stack([re1_b, re2_b, re3_b, re4_b]).reshape(4, 1, _P * _C)
    yo = pl.pallas_call(
        _rex_kernel,
        grid=(4, _B),
        in_specs=[pl.BlockSpec((1, 1, _N, _C * _RK), lambda l, b: (l, b, 0, 0)),
                  pl.BlockSpec((1, _C * _RK, _P * _C), lambda l, b: (l, 0, 0)),
                  pl.BlockSpec((1, 1, _P * _C), lambda l, b: (l, 0, 0))],
        out_specs=pl.BlockSpec((1, 1, _N, _P * _C), lambda l, b: (l, b, 0, 0)),
        out_shape=jax.ShapeDtypeStruct((4, _B, _N, _P * _C), BF16),
        compiler_params=pltpu.CompilerParams(
            dimension_semantics=("parallel", "parallel")),
    )(xt, wt, bt)
    projs = yo.reshape(4, _B, _N, _P, _C).transpose(0, 1, 4, 2, 3)
    p1, p2, p3, p4 = projs[0], projs[1], projs[2], projs[3]

    w64 = jnp.concatenate([conv1_w[:, :, 0, :], conv2_w[:, :, 0, :]],
                          axis=0)
    w_all = w64.transpose(2, 0, 1).reshape(_KW * 64, _G * _C).astype(BF16)
    bc = jnp.concatenate([conv1_b, conv2_b]).reshape(1, 1, 64)
    rw = res_w[:, :, 0, 0]
    rb = res_b.reshape(1, 1, _C)
    sw = skip_w[:, :, 0, 0]
    sb = skip_b.reshape(1, 1, _C)
    r4 = lambda a, t: a.reshape(_B, _C, _N, t)
    ins = [x.astype(BF16), r4(x1, _T), r4(ltm, _TE), r4(lts, _TE),
           r4(x2, _T), r4(sem, _TE), r4(ses, _TE),
           r4(x3, _T), r4(stm, _TE), r4(sts, _TE),
           r4(x4, _T), r4(snm, _TE), r4(sns, _TE), p1, p2, p3, p4]
    blk = lambda t: pl.BlockSpec((1, _C, _NB, t), lambda b, n: (b, 0, n, 0))
    wfull = lambda *s: pl.BlockSpec(s, lambda b, n: (0,) * len(s))
    xz, s = pl.pallas_call(
        _conv_kernel,
        grid=(_B, _N // _NB),
        in_specs=[blk(_T), blk(_T), blk(_TE), blk(_TE),
                  blk(_T), blk(_TE), blk(_TE),
                  blk(_T), blk(_TE), blk(_TE),
                  blk(_T), blk(_TE), blk(_TE),
                  blk(_P), blk(_P), blk(_P), blk(_P),
                  wfull(_KW * 64, _G * _C), wfull(1, 1, 64),
                  wfull(_C, _C), wfull(1, 1, _C),
                  wfull(_C, _C), wfull(1, 1, _C)],
        out_specs=[blk(_TO - _P), blk(_P)],
        out_shape=[jax.ShapeDtypeStruct((_B, _C, _N, _TO - _P), F32),
                   jax.ShapeDtypeStruct((_B, _C, _N, _P), F32)],
        compiler_params=pltpu.CompilerParams(
            dimension_semantics=("parallel", "parallel"),
            vmem_limit_bytes=50 * 1024 * 1024),
    )(*ins, w_all, bc, rw, rb, sw, sb)

    sel = np.concatenate(
        [np.arange(g * _C, (g + 1) * _C) for g in (2, 3, 5, 6, 8, 9, 11, 12)])
    wa = w64[:, sel, :].transpose(2, 0, 1).reshape(_KW * 64, 8 * _C)
    rt = lambda a: a.reshape(_B, _C, tw, _N)
    ttspec = pl.BlockSpec((1, _C, tw, _N), lambda b: (b, 0, 0, 0))
    saux = pl.pallas_call(
        _aux_kernel,
        grid=(_B,),
        in_specs=[ttspec] * 8 +
                 [pl.BlockSpec((_KW * 64, 8 * _C), lambda b: (0, 0)),
                  pl.BlockSpec((1, 1, 64), lambda b: (0, 0, 0)),
                  pl.BlockSpec((_C, _C), lambda b: (0, 0)),
                  pl.BlockSpec((1, 1, _C), lambda b: (0, 0, 0))],
        out_specs=pl.BlockSpec((1, _C, _N, _P), lambda b: (b, 0, 0, 0)),
        out_shape=jax.ShapeDtypeStruct((_B, _C, _N, _P), F32),
        compiler_params=pltpu.CompilerParams(
            dimension_semantics=("parallel",)),
    )(rt(tltm), rt(tlts), rt(tsem), rt(tses), rt(tstm), rt(tsts),
      rt(tsnm), rt(tsns), wa, bc, sw, sb)

    return xz, saux, s


# speedup vs baseline: 4.3346x; 1.0279x over previous
import jax
import jax.numpy as jnp
import numpy as np
from jax.experimental import pallas as pl
from jax.experimental.pallas import tpu as pltpu

F32 = jnp.float32
_B, _C, _N, _T = 4, 32, 256, 288
_P = 12
_KW = 12
_PERIOD, _SHORT, _LONG = 24, 12, 48
_RK = 5
_TE = _T + _P
_TO = _TE + 1 - _KW + 1
_NB = 16
BF16 = jnp.bfloat16
_G = 13


def _win_matrix(L):
    s = np.arange(_T)[:, None]
    t = np.maximum(np.arange(_T)[None, :], L - 1)
    return (((s <= t) & (s >= t - L + 1)).astype(np.float32) / L)


def _season_matrix(P):
    s = np.arange(_T)[:, None]
    t = np.arange(_T)[None, :]
    return ((s % P) == (t % P)).astype(np.float32) * (P / _T)


_M48 = _win_matrix(_LONG)
_S24 = _season_matrix(_PERIOD)
_M12 = _win_matrix(_SHORT)


def _adj_kernel(emb_ref, adj_ref):
    e = emb_ref[...]
    logits = jax.lax.dot_general(e, e, (((1,), (1,)), ((), ())),
                                 preferred_element_type=F32)
    r = jax.lax.broadcasted_iota(jnp.int32, (_N, _N), 0)
    c = jax.lax.broadcasted_iota(jnp.int32, (_N, _N), 1)
    logits = logits - jnp.where(r == c, 10.0, 0.0).astype(F32)
    logits = logits - jnp.max(logits, axis=-1, keepdims=True)
    p = jnp.exp(logits)
    adj_ref[...] = (p / jnp.sum(p, axis=-1, keepdims=True)).astype(BF16)


def _chain_kernel(x_ref, m48_ref, s24_ref, m12_ref, adj_ref,
                  x1_ref, ltm_ref, lts_ref,
                  x2_ref, sem_ref, ses_ref,
                  x3_ref, stm_ref, sts_ref,
                  x4_ref, snm_ref, sns_ref,
                  tltm_ref, tlts_ref, tsem_ref, tses_ref,
                  tstm_ref, tsts_ref, tsnm_ref, tsns_ref):
    xb = x_ref[0]

    def tnorm(v, M):
        m = jnp.dot(v.astype(BF16), M, preferred_element_type=F32)
        q = jnp.dot((v * v).astype(BF16), M, preferred_element_type=F32)
        var = q - m * m + 1e-5
        out = (v - m) * jax.lax.rsqrt(var + 1.0)
        return out, m, jnp.sqrt(var)

    x1, m1, s1 = tnorm(xb, m48_ref[...])
    x2, m2, s2 = tnorm(x1, s24_ref[...])
    x3, m3, s3 = tnorm(x2, m12_ref[...])

    a = adj_ref[...]
    ms = jnp.dot(a, x3.astype(BF16), preferred_element_type=F32)
    qs = jnp.dot(a, (x3 * x3).astype(BF16), preferred_element_type=F32)
    vs = qs - ms * ms + 1e-5
    x4 = (x3 - ms) * jax.lax.rsqrt(vs + 1.0)
    ss = jnp.sqrt(vs)

    def extc(m):
        return jnp.concatenate(
            [m, jnp.broadcast_to(m[:, _T - 1:_T], (_N, _P))], axis=-1)

    def exts(m):
        return jnp.concatenate([m, m[:, :_P]], axis=-1)

    x1_ref[0] = x1.astype(BF16)
    x2_ref[0] = x2.astype(BF16)
    x3_ref[0] = x3.astype(BF16)
    x4_ref[0] = x4.astype(BF16)
    tw0 = _TE - (_KW + _P - 1)
    for eref, tref, stat in ((ltm_ref, tltm_ref, extc(m1)),
                             (lts_ref, tlts_ref, extc(s1)),
                             (sem_ref, tsem_ref, exts(m2)),
                             (ses_ref, tses_ref, exts(s2)),
                             (stm_ref, tstm_ref, extc(m3)),
                             (sts_ref, tsts_ref, extc(s3)),
                             (snm_ref, tsnm_ref, extc(ms)),
                             (sns_ref, tsns_ref, extc(ss))):
        eref[0] = stat.astype(BF16)
        tref[0] = stat[:, tw0:].T


def _rex_kernel(x_ref, w_ref, b_ref, o_ref):
    o_ref[0, 0] = (jnp.dot(x_ref[0, 0], w_ref[0],
                           preferred_element_type=F32)
                   + b_ref[0, 0]).astype(BF16)


def _conv_kernel(x_ref, x1_ref, ltm_ref, lts_ref, x2_ref, sem_ref, ses_ref,
                 x3_ref, stm_ref, sts_ref, x4_ref, snm_ref, sns_ref,
                 p1_ref, p2_ref, p3_ref, p4_ref,
                 w_ref, bc_ref, rw_ref, rb_ref, sw_ref, sb_ref,
                 xz_ref, s_ref):
    def body(ref):
        b = ref[0].astype(BF16)
        return jnp.concatenate(
            [b, jnp.broadcast_to(b[:, :, _T - 1:_T], (_C, _NB, _P))], axis=-1)

    def withp(ref, pref):
        return jnp.concatenate([ref[0], pref[0]], axis=-1)

    groups = [body(x_ref), withp(x1_ref, p1_ref), ltm_ref[0], lts_ref[0],
              withp(x2_ref, p2_ref), sem_ref[0], ses_ref[0],
              withp(x3_ref, p3_ref), stm_ref[0], sts_ref[0],
              withp(x4_ref, p4_ref), snm_ref[0], sns_ref[0]]
    slab = jnp.concatenate(groups, axis=0)
    slab = jnp.concatenate(
        [jnp.zeros((_G * _C, _NB, 1), BF16), slab], axis=-1)

    y = jnp.einsum('oc,cnt->ont', w_ref[...], slab,
                   preferred_element_type=F32)
    acc = jnp.zeros((64, _NB, _TO), F32)
    for k in range(_KW):
        acc = acc + y[64 * k:64 * (k + 1), :, k:k + _TO]
    acc = acc + bc_ref[0, 0][:, None, None]

    prod = acc[:_C] * acc[_C:]
    xz = jnp.einsum('oc,cnt->ont', rw_ref[...], prod[:, :, :_TO - _P],
                    preferred_element_type=F32) + rb_ref[0, 0][:, None, None]
    sk = jnp.einsum('oc,cnt->ont', sw_ref[...], prod[:, :, _TO - _P:],
                    preferred_element_type=F32) + sb_ref[0, 0][:, None, None]
    xz_ref[0] = xz
    s_ref[0] = sk


def _aux_kernel(t1_ref, t2_ref, t3_ref, t4_ref, t5_ref, t6_ref, t7_ref,
                t8_ref, w_ref, bc_ref, sw_ref, sb_ref, o_ref):
    slab = jnp.concatenate(
        [r[0] for r in (t1_ref, t2_ref, t3_ref, t4_ref,
                        t5_ref, t6_ref, t7_ref, t8_ref)], axis=0)
    y = jnp.einsum('oc,ctn->otn', w_ref[...], slab,
                   preferred_element_type=F32)
    acc = jnp.zeros((64, _P, _N), F32)
    for k in range(_KW):
        acc = acc + y[64 * k:64 * (k + 1), k:k + _P, :]
    acc = acc + bc_ref[0, 0][:, None, None]
    prod = acc[:_C] * acc[_C:]
    res = jnp.einsum('oc,ctn->otn', sw_ref[...], prod,
                     preferred_element_type=F32) + sb_ref[0, 0][:, None, None]
    o_ref[0] = res.transpose(0, 2, 1)


def kernel(x, node_embedding, conv1_w, conv1_b, conv2_w, conv2_b,
           skip_w, skip_b, res_w, res_b,
           re1_w, re1_b, re2_w, re2_b, re3_w, re3_b, re4_w, re4_b):
    adj = pl.pallas_call(
        _adj_kernel,
        out_shape=jax.ShapeDtypeStruct((_N, _N), BF16),
    )(node_embedding)

    xf = x.reshape(_B * _C, _N, _T)
    big = jax.ShapeDtypeStruct((_B * _C, _N, _T), BF16)
    ext = jax.ShapeDtypeStruct((_B * _C, _N, _TE), BF16)
    row = lambda t: pl.BlockSpec((1, _N, t), lambda i: (i, 0, 0))
    full2 = lambda a, b: pl.BlockSpec((a, b), lambda i: (0, 0))
    tw = _KW + _P - 1
    tl = jax.ShapeDtypeStruct((_B * _C, tw, _N), F32)
    tspec = pl.BlockSpec((1, tw, _N), lambda i: (i, 0, 0))
    outs = pl.pallas_call(
        _chain_kernel,
        grid=(_B * _C,),
        in_specs=[row(_T), full2(_T, _T), full2(_T, _T), full2(_T, _T),
                  full2(_N, _N)],
        out_specs=[row(_T), row(_TE), row(_TE),
                   row(_T), row(_TE), row(_TE),
                   row(_T), row(_TE), row(_TE),
                   row(_T), row(_TE), row(_TE)] + [tspec] * 8,
        out_shape=[big, ext, ext, big, ext, ext, big, ext, ext,
                   big, ext, ext] + [tl] * 8,
        compiler_params=pltpu.CompilerParams(
            dimension_semantics=("parallel",)),
    )(xf, jnp.asarray(_M48, BF16), jnp.asarray(_S24, BF16),
      jnp.asarray(_M12, BF16), adj)
    (x1, ltm, lts, x2, sem, ses, x3, stm, sts, x4, snm, sns,
     tltm, tlts, tsem, tses, tstm, tsts, tsnm, tsns) = outs

    tails = jnp.stack([x1, x2, x3, x4]).reshape(
        4, _B, _C, _N, _T)[..., _T - _RK:]
    xt = tails.transpose(0, 1, 3, 2, 4).reshape(4, _B, _N, _C * _RK)
    wt = jnp.stack([re1_w, re2_w, re3_w, re4_w])[:, :, :, 0, :].transpose(
        0, 2, 3, 1).reshape(4, _C * _RK, _P * _C).astype(BF16)
    bt = jnp.stack([re1_b, re2_b, re3_b, re4_b]).reshape(4, 1, _P * _C)
    yo = pl.pallas_call(
        _rex_kernel,
        grid=(4, _B),
        in_specs=[pl.BlockSpec((1, 1, _N, _C * _RK), lambda l, b: (l, b, 0, 0)),
                  pl.BlockSpec((1, _C * _RK, _P * _C), lambda l, b: (l, 0, 0)),
                  pl.BlockSpec((1, 1, _P * _C), lambda l, b: (l, 0, 0))],
        out_specs=pl.BlockSpec((1, 1, _N, _P * _C), lambda l, b: (l, b, 0, 0)),
        out_shape=jax.ShapeDtypeStruct((4, _B, _N, _P * _C), BF16),
        compiler_params=pltpu.CompilerParams(
            dimension_semantics=("parallel", "parallel")),
    )(xt, wt, bt)
    projs = yo.reshape(4, _B, _N, _P, _C).transpose(0, 1, 4, 2, 3)
    p1, p2, p3, p4 = projs[0], projs[1], projs[2], projs[3]

    w64 = jnp.concatenate([conv1_w[:, :, 0, :], conv2_w[:, :, 0, :]],
                          axis=0)
    w_all = w64.transpose(2, 0, 1).reshape(_KW * 64, _G * _C).astype(BF16)
    bc = jnp.concatenate([conv1_b, conv2_b]).reshape(1, 1, 64)
    rw = res_w[:, :, 0, 0]
    rb = res_b.reshape(1, 1, _C)
    sw = skip_w[:, :, 0, 0]
    sb = skip_b.reshape(1, 1, _C)
    r4 = lambda a, t: a.reshape(_B, _C, _N, t)
    ins = [x, r4(x1, _T), r4(ltm, _TE), r4(lts, _TE),
           r4(x2, _T), r4(sem, _TE), r4(ses, _TE),
           r4(x3, _T), r4(stm, _TE), r4(sts, _TE),
           r4(x4, _T), r4(snm, _TE), r4(sns, _TE), p1, p2, p3, p4]
    blk = lambda t: pl.BlockSpec((1, _C, _NB, t), lambda b, n: (b, 0, n, 0))
    wfull = lambda *s: pl.BlockSpec(s, lambda b, n: (0,) * len(s))
    xz, s = pl.pallas_call(
        _conv_kernel,
        grid=(_B, _N // _NB),
        in_specs=[blk(_T), blk(_T), blk(_TE), blk(_TE),
                  blk(_T), blk(_TE), blk(_TE),
                  blk(_T), blk(_TE), blk(_TE),
                  blk(_T), blk(_TE), blk(_TE),
                  blk(_P), blk(_P), blk(_P), blk(_P),
                  wfull(_KW * 64, _G * _C), wfull(1, 1, 64),
                  wfull(_C, _C), wfull(1, 1, _C),
                  wfull(_C, _C), wfull(1, 1, _C)],
        out_specs=[blk(_TO - _P), blk(_P)],
        out_shape=[jax.ShapeDtypeStruct((_B, _C, _N, _TO - _P), F32),
                   jax.ShapeDtypeStruct((_B, _C, _N, _P), F32)],
        compiler_params=pltpu.CompilerParams(
            dimension_semantics=("parallel", "parallel"),
            vmem_limit_bytes=50 * 1024 * 1024),
    )(*ins, w_all, bc, rw, rb, sw, sb)

    sel = np.concatenate(
        [np.arange(g * _C, (g + 1) * _C) for g in (2, 3, 5, 6, 8, 9, 11, 12)])
    wa = w64[:, sel, :].transpose(2, 0, 1).reshape(_KW * 64, 8 * _C)
    rt = lambda a: a.reshape(_B, _C, tw, _N)
    ttspec = pl.BlockSpec((1, _C, tw, _N), lambda b: (b, 0, 0, 0))
    saux = pl.pallas_call(
        _aux_kernel,
        grid=(_B,),
        in_specs=[ttspec] * 8 +
                 [pl.BlockSpec((_KW * 64, 8 * _C), lambda b: (0, 0)),
                  pl.BlockSpec((1, 1, 64), lambda b: (0, 0, 0)),
                  pl.BlockSpec((_C, _C), lambda b: (0, 0)),
                  pl.BlockSpec((1, 1, _C), lambda b: (0, 0, 0))],
        out_specs=pl.BlockSpec((1, _C, _N, _P), lambda b: (b, 0, 0, 0)),
        out_shape=jax.ShapeDtypeStruct((_B, _C, _N, _P), F32),
        compiler_params=pltpu.CompilerParams(
            dimension_semantics=("parallel",)),
    )(rt(tltm), rt(tlts), rt(tsem), rt(tses), rt(tstm), rt(tsts),
      rt(tsnm), rt(tsns), wa, bc, sw, sb)

    return xz, saux, s
